# baseline (speedup 1.0000x reference)
import functools

import jax
import jax.numpy as jnp
from jax import lax
from jax.experimental import pallas as pl
from jax.experimental.pallas import tpu as pltpu
from jax.experimental.pallas import tpu_sc as plsc

NC = 2
NS = 16
NW = NC * NS
LANES = 16
NBUF = 4


def _sc_embed(x2d, token_table, pos_table):
    bsz, t_len = x2d.shape
    _, emb = token_table.shape

    rows_w = bsz // NW
    chunk = t_len
    n_chunk = rows_w
    vecs_per_row = emb // LANES

    sub_sizes = []
    off = 0
    while off < chunk:
        n = min(128, chunk - off)
        sub_sizes.append((off, n))
        off += n

    mesh = plsc.VectorSubcoreMesh(core_axis_name="c", subcore_axis_name="s")

    @functools.partial(
        pl.kernel,
        mesh=mesh,
        compiler_params=pltpu.CompilerParams(use_tc_tiling_on_sc=False),
        out_type=jax.ShapeDtypeStruct((bsz, t_len, emb), jnp.float32),
        scratch_types=[
            pltpu.VMEM((rows_w, t_len), jnp.int32),
            pltpu.VMEM((t_len, emb), jnp.float32),
            [pltpu.VMEM((chunk, emb), jnp.float32) for _ in range(NBUF)],
            [pltpu.SemaphoreType.DMA for _ in range(NBUF)],
            [pltpu.SemaphoreType.DMA for _ in range(NBUF)],
        ],
    )
    def k(idx_hbm, tok_hbm, pos_hbm, out_hbm, idx_v, pos_v, bufs, gsems, ssems):
        wid = lax.axis_index("s") * NC + lax.axis_index("c")
        base_row = wid * rows_w
        pltpu.sync_copy(idx_hbm.at[pl.ds(base_row, rows_w)], idx_v)
        pltpu.sync_copy(pos_hbm, pos_v)

        def g_issue(r, buf, sem):
            for so, sn in sub_sizes:
                pltpu.async_copy(
                    tok_hbm.at[idx_v.at[r, pl.ds(so, sn)]],
                    buf.at[pl.ds(so, sn)], sem)

        def g_wait(r, buf, sem):
            for so, sn in sub_sizes:
                pltpu.make_async_copy(
                    tok_hbm.at[idx_v.at[r, pl.ds(so, sn)]],
                    buf.at[pl.ds(so, sn)], sem).wait()

        def s_issue(r, buf, sem):
            pltpu.async_copy(buf, out_hbm.at[base_row + r], sem)

        def s_wait(r, buf, sem):
            pltpu.make_async_copy(buf, out_hbm.at[base_row + r], sem).wait()

        def add_pos(buf):
            @plsc.parallel_loop(0, chunk, 1, unroll=8)
            def _(j):
                for v in range(vecs_per_row):
                    sl = pl.ds(v * LANES, LANES)
                    plsc.addupdate(buf.at[j, sl], pos_v[j, sl])

        g_issue(0, bufs[0], gsems[0])
        g_issue(1, bufs[1], gsems[1])

        def macro(m, carry):
            for i in range(NBUF):
                g = NBUF * m + i
                g_wait(g, bufs[i], gsems[i])
                add_pos(bufs[i])
                s_issue(g, bufs[i], ssems[i])
                i2 = (i + 2) % NBUF
                g2 = g + 2

                @pl.when(g2 < n_chunk)
                def _():
                    @pl.when(g2 >= NBUF)
                    def _():
                        s_wait(g - 2, bufs[i2], ssems[i2])
                    g_issue(g2, bufs[i2], gsems[i2])
            return carry

        lax.fori_loop(0, n_chunk // NBUF, macro, 0)

        for i in range(NBUF):
            s_wait(n_chunk - NBUF + i, bufs[i], ssems[i])

    return k(x2d, token_table, pos_table)


def kernel(x, token_table, pos_table):
    return _sc_embed(x.astype(jnp.int32), token_table, pos_table)


# speedup vs baseline: 1.3275x; 1.3275x over previous
import functools

import jax
import jax.numpy as jnp
from jax import lax
from jax.experimental import pallas as pl
from jax.experimental.pallas import tpu as pltpu
from jax.experimental.pallas import tpu_sc as plsc

NC = 2
NS = 16
NW = NC * NS
LANES = 16
NBUF = 4


def _sc_embed(x2d, token_table, pos_table):
    bsz, t_len = x2d.shape
    _, emb = token_table.shape

    rows_w = bsz // NW
    chunk = t_len
    n_chunk = rows_w
    vecs_per_row = emb // LANES

    sub_sizes = []
    off = 0
    while off < chunk:
        n = min(128, chunk - off)
        sub_sizes.append((off, n))
        off += n

    mesh = plsc.VectorSubcoreMesh(core_axis_name="c", subcore_axis_name="s")

    @functools.partial(
        pl.kernel,
        mesh=mesh,
        compiler_params=pltpu.CompilerParams(use_tc_tiling_on_sc=False),
        out_type=jax.ShapeDtypeStruct((bsz, t_len, 2 * emb), jnp.float32),
        scratch_types=[
            pltpu.VMEM((rows_w, t_len), jnp.int32),
            pltpu.VMEM((t_len, emb), jnp.float32),
            [pltpu.VMEM((chunk, emb), jnp.float32) for _ in range(NBUF)],
            [pltpu.SemaphoreType.DMA for _ in range(NBUF)],
            [pltpu.SemaphoreType.DMA for _ in range(NBUF)],
        ],
    )
    def k(idx_hbm, tok_hbm, pos_hbm, out_hbm, idx_v, pos_v, bufs, gsems, ssems):
        wid = lax.axis_index("s") * NC + lax.axis_index("c")
        base_row = wid * rows_w
        pltpu.sync_copy(idx_hbm.at[pl.ds(base_row, rows_w)], idx_v)
        pltpu.sync_copy(pos_hbm, pos_v)

        def g_issue(r, buf, sem):
            for so, sn in sub_sizes:
                pltpu.async_copy(
                    tok_hbm.at[idx_v.at[r, pl.ds(so, sn)]],
                    buf.at[pl.ds(so, sn)], sem)

        def g_wait(r, buf, sem):
            for so, sn in sub_sizes:
                pltpu.make_async_copy(
                    tok_hbm.at[idx_v.at[r, pl.ds(so, sn)]],
                    buf.at[pl.ds(so, sn)], sem).wait()

        def s_issue(r, buf, sem):
            pltpu.async_copy(
                buf, out_hbm.at[base_row + r, pl.ds(0, t_len), pl.ds(0, emb)],
                sem)

        def s_wait(r, buf, sem):
            pltpu.make_async_copy(
                buf, out_hbm.at[base_row + r, pl.ds(0, t_len), pl.ds(0, emb)],
                sem).wait()

        def add_pos(buf):
            @plsc.parallel_loop(0, chunk, 1, unroll=8)
            def _(j):
                for v in range(vecs_per_row):
                    sl = pl.ds(v * LANES, LANES)
                    plsc.addupdate(buf.at[j, sl], pos_v[j, sl])

        g_issue(0, bufs[0], gsems[0])
        g_issue(1, bufs[1], gsems[1])

        def macro(m, carry):
            for i in range(NBUF):
                g = NBUF * m + i
                g_wait(g, bufs[i], gsems[i])
                add_pos(bufs[i])
                s_issue(g, bufs[i], ssems[i])
                i2 = (i + 2) % NBUF
                g2 = g + 2

                @pl.when(g2 < n_chunk)
                def _():
                    @pl.when(g2 >= NBUF)
                    def _():
                        s_wait(g - 2, bufs[i2], ssems[i2])
                    g_issue(g2, bufs[i2], gsems[i2])
            return carry

        lax.fori_loop(0, n_chunk // NBUF, macro, 0)

        for i in range(NBUF):
            s_wait(n_chunk - NBUF + i, bufs[i], ssems[i])

    return k(x2d, token_table, pos_table)


def kernel(x, token_table, pos_table):
    emb = token_table.shape[1]
    out = _sc_embed(x.astype(jnp.int32), token_table, pos_table)
    return out[:, :, :emb]
